# Initial kernel scaffold; baseline (speedup 1.0000x reference)
import jax
import jax.numpy as jnp
from jax.experimental import pallas as pl


def kernel(bits, tables):
    raise NotImplementedError("write your pallas kernel here")


# speedup vs baseline: 5.5783x; 5.5783x over previous
import functools

import jax
import jax.numpy as jnp
from jax import lax
from jax.experimental import pallas as pl
from jax.experimental.pallas import tpu as pltpu
from jax.experimental.pallas import tpu_sc as plsc

N_BITS = 16
BATCH = 1024
MAX_TABLE = 1 << (N_BITS - 1)

NC = 2
NS = 16
NW = NC * NS
ROWS = BATCH // NW
GROUPS = ROWS // 16
IDX_MINOR = 128
GCHUNKS = ROWS * N_BITS // IDX_MINOR


def _mapper_body(bits_hbm, tabs_hbm, out_hbm, bits_v, idx_v, got_v, out_v, sem):
    wid = lax.axis_index("s") * NC + lax.axis_index("c")
    base = wid * ROWS

    pltpu.sync_copy(bits_hbm.at[pl.ds(base, ROWS), :], bits_v)

    lane = lax.iota(jnp.int32, 16)
    zero = lane * 0

    for g in range(GROUPS):
        rowsel = lane + g * 16
        d0 = 2 * g + (lane >> 3)
        d1base = (lane & 7) << 4
        addr = zero
        for p in range(N_BITS):
            flat = addr + p * MAX_TABLE
            plsc.store_scatter(idx_v, [d0, d1base + (15 - p)], flat)
            if p < N_BITS - 1:
                col = plsc.load_gather(bits_v, [rowsel, zero + (15 - p)])
                addr = addr + (col << p)

    copies = [
        pltpu.async_copy(tabs_hbm.at[idx_v.at[j]], got_v.at[j], sem)
        for j in range(GCHUNKS)
    ]
    for c in copies:
        c.wait()

    for r in range(ROWS):
        bf = bits_v[r, :].astype(jnp.float32)
        t = got_v[r // 8, pl.ds((r % 8) * 16, 16)]
        out_v[r, :] = bf + t - 2.0 * bf * t

    pltpu.sync_copy(out_v, out_hbm.at[pl.ds(base, ROWS), :])


@functools.cache
def _build_mapper():
    return functools.partial(
        pl.kernel,
        out_type=jax.ShapeDtypeStruct((BATCH, N_BITS), jnp.float32),
        mesh=plsc.VectorSubcoreMesh(core_axis_name="c", subcore_axis_name="s"),
        compiler_params=pltpu.CompilerParams(needs_layout_passes=False),
        scratch_types=[
            pltpu.VMEM((ROWS, N_BITS), jnp.int32),
            pltpu.VMEM((GCHUNKS, IDX_MINOR), jnp.int32),
            pltpu.VMEM((GCHUNKS, IDX_MINOR), jnp.float32),
            pltpu.VMEM((ROWS, N_BITS), jnp.float32),
            pltpu.SemaphoreType.DMA,
        ],
    )(_mapper_body)


def kernel(bits, tables):
    return _build_mapper()(bits, tables.reshape(-1))


# speedup vs baseline: 5.5989x; 1.0037x over previous
import functools

import jax
import jax.numpy as jnp
from jax import lax
from jax.experimental import pallas as pl
from jax.experimental.pallas import tpu as pltpu
from jax.experimental.pallas import tpu_sc as plsc

N_BITS = 16
BATCH = 1024
MAX_TABLE = 1 << (N_BITS - 1)

NC = 2
NS = 16
NW = NC * NS
ROWS = BATCH // NW
GROUPS = ROWS // 16
IDX_MINOR = 128
GCHUNKS = ROWS * N_BITS // IDX_MINOR


def _mapper_body(bits_hbm, tabs_hbm, out_hbm, bits_v, idx_v, got_v, out_v, sem):
    wid = lax.axis_index("s") * NC + lax.axis_index("c")
    base = wid * ROWS

    pltpu.sync_copy(bits_hbm.at[pl.ds(base, ROWS), :], bits_v)

    lane = lax.iota(jnp.int32, 16)
    zero = lane * 0

    for g in range(GROUPS):
        rowsel = lane + g * 16
        dbase = rowsel << 4
        addr = zero
        for p in range(N_BITS):
            flat = addr + p * MAX_TABLE
            plsc.store_scatter(idx_v, [dbase + (15 - p)], flat)
            if p < N_BITS - 1:
                col = plsc.load_gather(bits_v, [rowsel, zero + (15 - p)])
                addr = addr + (col << p)

    pltpu.async_copy(tabs_hbm.at[idx_v], got_v, sem).wait()

    for r in range(ROWS):
        bf = bits_v[r, :].astype(jnp.float32)
        t = got_v[pl.ds(r * 16, 16)]
        out_v[r, :] = bf + t - 2.0 * bf * t

    pltpu.sync_copy(out_v, out_hbm.at[pl.ds(base, ROWS), :])


@functools.cache
def _build_mapper():
    return functools.partial(
        pl.kernel,
        out_type=jax.ShapeDtypeStruct((BATCH, N_BITS), jnp.float32),
        mesh=plsc.VectorSubcoreMesh(core_axis_name="c", subcore_axis_name="s"),
        compiler_params=pltpu.CompilerParams(needs_layout_passes=False),
        scratch_types=[
            pltpu.VMEM((ROWS, N_BITS), jnp.int32),
            pltpu.VMEM((ROWS * N_BITS,), jnp.int32),
            pltpu.VMEM((ROWS * N_BITS,), jnp.float32),
            pltpu.VMEM((ROWS, N_BITS), jnp.float32),
            pltpu.SemaphoreType.DMA,
        ],
    )(_mapper_body)


def kernel(bits, tables):
    return _build_mapper()(bits, tables.reshape(-1))


# speedup vs baseline: 5.6655x; 1.0119x over previous
import functools

import jax
import jax.numpy as jnp
from jax import lax
from jax.experimental import pallas as pl
from jax.experimental.pallas import tpu as pltpu
from jax.experimental.pallas import tpu_sc as plsc

N_BITS = 16
BATCH = 1024
MAX_TABLE = 1 << (N_BITS - 1)

NC = 1
NS = 16
NW = NC * NS
ROWS = BATCH // NW
GROUPS = ROWS // 16
IDX_MINOR = 128
GCHUNKS = ROWS * N_BITS // IDX_MINOR


def _mapper_body(bits_hbm, tabs_hbm, out_hbm, bits_v, idx_v, got_v, out_v, sem):
    wid = lax.axis_index("s") * NC + lax.axis_index("c")
    base = wid * ROWS

    pltpu.sync_copy(bits_hbm.at[pl.ds(base, ROWS), :], bits_v)

    lane = lax.iota(jnp.int32, 16)
    zero = lane * 0

    for g in range(GROUPS):
        rowsel = lane + g * 16
        dbase = rowsel << 4
        addr = zero
        for p in range(N_BITS):
            flat = addr + p * MAX_TABLE
            plsc.store_scatter(idx_v, [dbase + (15 - p)], flat)
            if p < N_BITS - 1:
                col = plsc.load_gather(bits_v, [rowsel, zero + (15 - p)])
                addr = addr + (col << p)

    pltpu.async_copy(tabs_hbm.at[idx_v], got_v, sem).wait()

    for r in range(ROWS):
        bf = bits_v[r, :].astype(jnp.float32)
        t = got_v[pl.ds(r * 16, 16)]
        out_v[r, :] = bf + t - 2.0 * bf * t

    pltpu.sync_copy(out_v, out_hbm.at[pl.ds(base, ROWS), :])


@functools.cache
def _build_mapper():
    return functools.partial(
        pl.kernel,
        out_type=jax.ShapeDtypeStruct((BATCH, N_BITS), jnp.float32),
        mesh=plsc.VectorSubcoreMesh(
            core_axis_name="c", subcore_axis_name="s", num_cores=NC
        ),
        compiler_params=pltpu.CompilerParams(needs_layout_passes=False),
        scratch_types=[
            pltpu.VMEM((ROWS, N_BITS), jnp.int32),
            pltpu.VMEM((ROWS * N_BITS,), jnp.int32),
            pltpu.VMEM((ROWS * N_BITS,), jnp.float32),
            pltpu.VMEM((ROWS, N_BITS), jnp.float32),
            pltpu.SemaphoreType.DMA,
        ],
    )(_mapper_body)


def kernel(bits, tables):
    return _build_mapper()(bits, tables.reshape(-1))
